# baseline (speedup 1.0000x reference)
import functools

import jax
import jax.numpy as jnp
from jax import lax
from jax.experimental import pallas as pl
from jax.experimental.pallas import tpu as pltpu
from jax.experimental.pallas import tpu_sc as plsc

D = 262144
B = 64
L = 16


def _sc_permute(xt, lin):
    info = plsc.get_sparse_core_info()
    nw = info.num_cores * info.num_subcores
    S = D // nw

    mesh = plsc.VectorSubcoreMesh(core_axis_name="c", subcore_axis_name="s")

    @functools.partial(
        pl.kernel,
        mesh=mesh,
        out_type=jax.ShapeDtypeStruct((B, D), jnp.float32),
        scratch_types=[
            pltpu.VMEM((L,), jnp.int32),
            pltpu.SemaphoreType.DMA,
        ],
    )
    def k(x_hbm, lin_hbm, out_hbm, lbuf, sem):
        wid = lax.axis_index("s") * info.num_cores + lax.axis_index("c")
        c0 = wid * S
        pltpu.sync_copy(lin_hbm.at[pl.ds(c0, L)], lbuf)
        dst0 = pl.multiple_of(lbuf[...][0], S)
        c0 = pl.multiple_of(c0, S)
        pltpu.async_copy(
            x_hbm.at[:, pl.ds(c0, S)],
            out_hbm.at[:, pl.ds(dst0, S)],
            sem,
        ).wait()

    return k(xt, lin)


def kernel(x, lin):
    xt = x.T
    outt = _sc_permute(xt, lin.astype(jnp.int32))
    outc = jax.lax.complex(outt, jnp.zeros_like(outt))
    return outc.T


# speedup vs baseline: 2.7569x; 2.7569x over previous
import functools

import jax
import jax.numpy as jnp
from jax import lax
from jax.experimental import pallas as pl
from jax.experimental.pallas import tpu as pltpu
from jax.experimental.pallas import tpu_sc as plsc

D = 262144
B = 64
L = 16


def _sc_permute(xt, lin):
    info = plsc.get_sparse_core_info()
    nw = info.num_cores * info.num_subcores
    S = D // nw

    mesh = plsc.VectorSubcoreMesh(core_axis_name="c", subcore_axis_name="s")

    @functools.partial(
        pl.kernel,
        mesh=mesh,
        out_type=jax.ShapeDtypeStruct((B, D), jnp.float32),
        scratch_types=[
            pltpu.VMEM((L,), jnp.int32),
            pltpu.VMEM((B, 512), jnp.float32),
            pltpu.VMEM((B, 512), jnp.float32),
            pltpu.SemaphoreType.DMA,
            pltpu.SemaphoreType.DMA,
            pltpu.SemaphoreType.DMA,
            pltpu.SemaphoreType.DMA,
        ],
    )
    def k(x_hbm, lin_hbm, out_hbm, lbuf, buf0, buf1, li0, li1, so0, so1):
        wid = lax.axis_index("s") * info.num_cores + lax.axis_index("c")
        c0 = wid * S
        pltpu.sync_copy(lin_hbm.at[pl.ds(c0, L)], lbuf)
        dst0 = pl.multiple_of(lbuf[...][0], S)
        c0 = pl.multiple_of(c0, S)
        CW = 512
        n = S // CW
        bufs = (buf0, buf1)
        lsems = (li0, li1)
        osems = (so0, so1)

        def load(i, b):
            return pltpu.async_copy(
                x_hbm.at[:, pl.ds(c0 + i * CW, CW)], bufs[b], lsems[b])

        def store(i, b):
            return pltpu.async_copy(
                bufs[b], out_hbm.at[:, pl.ds(dst0 + i * CW, CW)], osems[b])

        load(0, 0).wait()
        for i in range(n):
            b = i & 1
            nxt = load(i + 1, 1 - b) if i + 1 < n else None
            store(i, b).wait()
            if nxt is not None:
                nxt.wait()

    return k(xt, lin)


def kernel(x, lin):
    xt = x.T
    outt = _sc_permute(xt, lin.astype(jnp.int32))
    outc = jax.lax.complex(outt, jnp.zeros_like(outt))
    return outc.T
